# baseline (speedup 1.0000x reference)
import functools

import jax
import jax.numpy as jnp
from jax import lax
from jax.experimental import pallas as pl
from jax.experimental.pallas import tpu as pltpu
from jax.experimental.pallas import tpu_sc as plsc

N = 10000
K = 32
KP = 15
C = 128
EXP = 4
SIGMA = 0.15

NPAD = 10240
B = 128
MPAD = NPAD * K
CH = 128

_f32 = jnp.float32


def _sc_gather1_body(tab, idx, out, idx_v, buf, gsems, dsems, *, nbuf, nc,
                     per_w, chunks):
  wid = lax.axis_index("s") * nc + lax.axis_index("c")
  base0 = wid * per_w
  pltpu.sync_copy(idx.at[pl.ds(base0, per_w)], idx_v)

  def issue(j, t):
    b = j * (nbuf * CH) + t * CH
    pltpu.async_copy(tab.at[idx_v.at[pl.ds(b, CH)]], buf.at[t], gsems[t])

  def wait_g(t):
    pltpu.make_async_copy(tab.at[idx_v.at[pl.ds(0, CH)]], buf.at[t],
                          gsems[t]).wait()

  def drain(j, t):
    b = j * (nbuf * CH) + t * CH
    pltpu.async_copy(buf.at[t], out.at[pl.ds(base0 + b, CH)], dsems[t])

  def wait_d(t):
    pltpu.make_async_copy(buf.at[t], out.at[pl.ds(base0, CH)],
                          dsems[t]).wait()

  def body(j, carry):
    for t in range(nbuf):
      wait_g(t)
      drain(j - 1, t)
    for t in range(nbuf):
      wait_d(t)
      issue(j, t)
    return carry

  for t in range(nbuf):
    issue(0, t)
  lax.fori_loop(1, chunks // nbuf, body, 0)
  for t in range(nbuf):
    wait_g(t)
    drain(chunks // nbuf - 1, t)
  for t in range(nbuf):
    wait_d(t)


def _make_sc_calls():
  info = plsc.get_sparse_core_info()
  nc, ns = info.num_cores, info.num_subcores
  per_w = MPAD // (nc * ns)
  chunks = per_w // CH
  mesh = plsc.VectorSubcoreMesh(core_axis_name="c", subcore_axis_name="s")

  def mk(body, width, nbuf, out_type):
    assert chunks % nbuf == 0
    return pl.kernel(
        functools.partial(body, nbuf=nbuf, nc=nc, per_w=per_w,
                          chunks=chunks),
        mesh=mesh,
        out_type=out_type,
        scratch_types=[
            pltpu.VMEM((per_w,), jnp.int32),
            pltpu.VMEM((nbuf, CH, width), _f32),
            [pltpu.SemaphoreType.DMA] * nbuf,
            [pltpu.SemaphoreType.DMA] * nbuf,
        ],
    )

  gather2 = mk(_sc_gather1_body, 2 * C, 2,
               jax.ShapeDtypeStruct((MPAD, 2 * C), _f32))
  gather1 = mk(_sc_gather1_body, C, 5,
               jax.ShapeDtypeStruct((MPAD, C), _f32))
  return gather2, gather1


def _leaky(x):
  return jnp.where(x >= 0, x, 0.1 * x)


def _infl_all(gp, ctr, kpt):
  dx = gp[:, :, 0:1] - ctr[:, :, 0:1]
  dy = gp[:, :, 1:2] - ctr[:, :, 1:2]
  dz = gp[:, :, 2:3] - ctr[:, :, 2:3]
  kx = kpt[0:1, :].reshape(1, 1, 16)
  ky = kpt[1:2, :].reshape(1, 1, 16)
  kz = kpt[2:3, :].reshape(1, 1, 16)
  ex = dx - kx
  ey = dy - ky
  ez = dz - kz
  d2 = ex * ex + ey * ey + ez * ez
  dist = jnp.sqrt(d2 + 1e-12)
  return jnp.maximum(1.0 - dist * (1.0 / SIGMA), 0.0)


def t1_body(gc_ref, pts_ref, kpt_ref, wst_ref, wa1_ref,
            x_ref, h1_ref, infl_ref):
  f = gc_ref[:, :, 0:C]
  gp = gc_ref[:, :, C:C + 16]
  infl = _infl_all(gp, pts_ref[...], kpt_ref[...])
  x = jnp.zeros((B, C), _f32)
  for p in range(KP):
    aggp = jnp.sum(f * infl[:, :, p:p + 1], axis=1)
    x = x + jnp.dot(aggp, wst_ref[p], preferred_element_type=_f32)
  x = _leaky(x)
  x_ref[...] = x
  h1_ref[...] = _leaky(jnp.dot(x, wa1_ref[...], preferred_element_type=_f32))
  infl_ref[...] = infl


def t23_body(gh_ref, infl_ref, x_ref, wdw_ref, wb_ref,
             wc_ref, wa_ref, x2_ref, h2_ref, *, last):
  infl = infl_ref[...]
  wedge = jnp.dot(infl.reshape(B * K, 16), wdw_ref[...],
                  preferred_element_type=_f32)
  g = gh_ref[...].reshape(B * K, C)
  h = jnp.sum((g * wedge).reshape(B, K, C), axis=1)
  h = _leaky(h)
  h = _leaky(jnp.dot(h, wb_ref[...], preferred_element_type=_f32))
  h = jnp.dot(h, wc_ref[...], preferred_element_type=_f32)
  x2 = x_ref[...] + h
  x2_ref[...] = x2
  if not last:
    h2_ref[...] = _leaky(jnp.dot(x2, wa_ref[...],
                                 preferred_element_type=_f32))


def _edge_spec():
  return pl.BlockSpec((B, K, C), lambda i: (i, 0, 0))


def _full(shape):
  return pl.BlockSpec(shape, lambda i: tuple(0 for _ in shape))


def _make_tc_calls():
  grid = (NPAD // B,)
  row_spec = pl.BlockSpec((B, C), lambda i: (i, 0))
  gp_spec = pl.BlockSpec((B, K, 16), lambda i: (i, 0, 0))
  pts_spec = pl.BlockSpec((B, 1, 3), lambda i: (i, 0, 0))

  t1 = pl.pallas_call(
      t1_body,
      grid=grid,
      in_specs=[
          pl.BlockSpec((B, K, 2 * C), lambda i: (i, 0, 0)), pts_spec,
          _full((8, 16)), _full((KP, C, C)), _full((C, C)),
      ],
      out_specs=[row_spec, row_spec, gp_spec],
      out_shape=[
          jax.ShapeDtypeStruct((NPAD, C), _f32),
          jax.ShapeDtypeStruct((NPAD, C), _f32),
          jax.ShapeDtypeStruct((NPAD, K, 16), _f32),
      ],
  )

  def make_t23(last):
    return pl.pallas_call(
        functools.partial(t23_body, last=last),
        grid=grid,
        in_specs=[
            _edge_spec(), gp_spec, row_spec,
            _full((16, C)), _full((C, EXP * C)), _full((EXP * C, C)),
            _full((C, C)),
        ],
        out_specs=[row_spec, row_spec],
        out_shape=[
            jax.ShapeDtypeStruct((NPAD, C), _f32),
            jax.ShapeDtypeStruct((NPAD, C), _f32),
        ],
    )

  return t1, make_t23(False), make_t23(True)


@jax.jit
def kernel(points, features, neighbors, kernel_points, W_stem,
           W_a1, W_dw1, W_b1, W_c1, W_a2, W_dw2, W_b2, W_c2):
  gather2, gather1 = _make_sc_calls()
  t1, t2, t3 = _make_tc_calls()

  ftab = jnp.pad(features, ((0, NPAD - N), (0, 0)))
  ptab = jnp.pad(points, ((0, NPAD - N), (0, 125)))
  ctab = jnp.concatenate([ftab, ptab], axis=1)
  idx = jnp.pad(neighbors, ((0, NPAD - N), (0, 0))).reshape(MPAD)
  pts3 = jnp.pad(points, ((0, NPAD - N), (0, 0))).reshape(NPAD, 1, 3)
  kpt = jnp.pad(kernel_points.T, ((0, 5), (0, 1)))
  wdw1 = jnp.pad(W_dw1, ((0, 1), (0, 0)))
  wdw2 = jnp.pad(W_dw2, ((0, 1), (0, 0)))

  gc = gather2(ctab, idx).reshape(NPAD, K, 2 * C)

  x1, h1, infl = t1(gc, pts3, kpt, W_stem, W_a1)

  g1 = gather1(h1, idx).reshape(NPAD, K, C)
  x2, h2 = t2(g1, infl, x1, wdw1, W_b1, W_c1, W_a2)

  g2 = gather1(h2, idx).reshape(NPAD, K, C)
  x3, _ = t3(g2, infl, x2, wdw2, W_b2, W_c2, W_a2)

  return x3[:N]


# speedup vs baseline: 1.0330x; 1.0330x over previous
import functools

import jax
import jax.numpy as jnp
from jax import lax
from jax.experimental import pallas as pl
from jax.experimental.pallas import tpu as pltpu
from jax.experimental.pallas import tpu_sc as plsc

N = 10000
K = 32
KP = 15
C = 128
EXP = 4
SIGMA = 0.15

NPAD = 10240
B = 256
MPAD = NPAD * K
CH = 128

_f32 = jnp.float32


def _sc_gather1_body(tab, idx, out, idx_v, buf, gsems, *, nbuf, nc,
                     per_w, chunks):
  wid = lax.axis_index("s") * nc + lax.axis_index("c")
  base0 = wid * per_w
  pltpu.sync_copy(idx.at[pl.ds(base0, per_w)], idx_v)

  def issue(j, t):
    b = j * (nbuf * CH) + t * CH
    pltpu.async_copy(tab.at[idx_v.at[pl.ds(b, CH)]], buf.at[t], gsems[t])

  def wait_g(t):
    pltpu.make_async_copy(tab.at[idx_v.at[pl.ds(0, CH)]], buf.at[t],
                          gsems[t]).wait()

  def drain(j, t):
    b = j * (nbuf * CH) + t * CH
    pltpu.sync_copy(buf.at[t], out.at[pl.ds(base0 + b, CH)])

  def body(j, carry):
    for t in range(nbuf):
      wait_g(t)
      drain(j - 1, t)
      issue(j, t)
    return carry

  for t in range(nbuf):
    issue(0, t)
  lax.fori_loop(1, chunks // nbuf, body, 0)
  for t in range(nbuf):
    wait_g(t)
    drain(chunks // nbuf - 1, t)


def _make_sc_calls():
  info = plsc.get_sparse_core_info()
  nc, ns = info.num_cores, info.num_subcores
  per_w = MPAD // (nc * ns)
  chunks = per_w // CH
  mesh = plsc.VectorSubcoreMesh(core_axis_name="c", subcore_axis_name="s")

  def mk(body, width, nbuf, out_type):
    assert chunks % nbuf == 0
    return pl.kernel(
        functools.partial(body, nbuf=nbuf, nc=nc, per_w=per_w,
                          chunks=chunks),
        mesh=mesh,
        out_type=out_type,
        scratch_types=[
            pltpu.VMEM((per_w,), jnp.int32),
            pltpu.VMEM((nbuf, CH, width), _f32),
            [pltpu.SemaphoreType.DMA] * nbuf,
        ],
    )

  gather2 = mk(_sc_gather1_body, 2 * C, 2,
               jax.ShapeDtypeStruct((MPAD, 2 * C), _f32))
  gather1 = mk(_sc_gather1_body, C, 5,
               jax.ShapeDtypeStruct((MPAD, C), _f32))
  return gather2, gather1


def _leaky(x):
  return jnp.where(x >= 0, x, 0.1 * x)


def _infl_all(gp, ctr, kpt):
  dx = gp[:, :, 0:1] - ctr[:, :, 0:1]
  dy = gp[:, :, 1:2] - ctr[:, :, 1:2]
  dz = gp[:, :, 2:3] - ctr[:, :, 2:3]
  kx = kpt[0:1, :].reshape(1, 1, 16)
  ky = kpt[1:2, :].reshape(1, 1, 16)
  kz = kpt[2:3, :].reshape(1, 1, 16)
  ex = dx - kx
  ey = dy - ky
  ez = dz - kz
  d2 = ex * ex + ey * ey + ez * ez
  dist = jnp.sqrt(d2 + 1e-12)
  return jnp.maximum(1.0 - dist * (1.0 / SIGMA), 0.0)


def t1_body(gc_ref, pts_ref, kpt_ref, wst_ref, wa1_ref,
            x_ref, h1_ref, infl_ref):
  f = gc_ref[:, :, 0:C]
  gp = gc_ref[:, :, C:C + 16]
  infl = _infl_all(gp, pts_ref[...], kpt_ref[...])
  x = jnp.zeros((B, C), _f32)
  for p in range(KP):
    aggp = jnp.sum(f * infl[:, :, p:p + 1], axis=1)
    x = x + jnp.dot(aggp, wst_ref[p], preferred_element_type=_f32)
  x = _leaky(x)
  x_ref[...] = x
  h1_ref[...] = _leaky(jnp.dot(x, wa1_ref[...], preferred_element_type=_f32))
  infl_ref[...] = infl


def t23_body(gh_ref, infl_ref, x_ref, wdw_ref, wb_ref,
             wc_ref, wa_ref, x2_ref, h2_ref, *, last):
  infl = infl_ref[...]
  wedge = jnp.dot(infl.reshape(B * K, 16), wdw_ref[...],
                  preferred_element_type=_f32)
  g = gh_ref[...].reshape(B * K, C)
  h = jnp.sum((g * wedge).reshape(B, K, C), axis=1)
  h = _leaky(h)
  h = _leaky(jnp.dot(h, wb_ref[...], preferred_element_type=_f32))
  h = jnp.dot(h, wc_ref[...], preferred_element_type=_f32)
  x2 = x_ref[...] + h
  x2_ref[...] = x2
  if not last:
    h2_ref[...] = _leaky(jnp.dot(x2, wa_ref[...],
                                 preferred_element_type=_f32))


def _edge_spec():
  return pl.BlockSpec((B, K, C), lambda i: (i, 0, 0))


def _full(shape):
  return pl.BlockSpec(shape, lambda i: tuple(0 for _ in shape))


def _make_tc_calls():
  grid = (NPAD // B,)
  row_spec = pl.BlockSpec((B, C), lambda i: (i, 0))
  gp_spec = pl.BlockSpec((B, K, 16), lambda i: (i, 0, 0))
  pts_spec = pl.BlockSpec((B, 1, 3), lambda i: (i, 0, 0))

  t1 = pl.pallas_call(
      t1_body,
      grid=grid,
      in_specs=[
          pl.BlockSpec((B, K, 2 * C), lambda i: (i, 0, 0)), pts_spec,
          _full((8, 16)), _full((KP, C, C)), _full((C, C)),
      ],
      out_specs=[row_spec, row_spec, gp_spec],
      out_shape=[
          jax.ShapeDtypeStruct((NPAD, C), _f32),
          jax.ShapeDtypeStruct((NPAD, C), _f32),
          jax.ShapeDtypeStruct((NPAD, K, 16), _f32),
      ],
  )

  def make_t23(last):
    return pl.pallas_call(
        functools.partial(t23_body, last=last),
        grid=grid,
        in_specs=[
            _edge_spec(), gp_spec, row_spec,
            _full((16, C)), _full((C, EXP * C)), _full((EXP * C, C)),
            _full((C, C)),
        ],
        out_specs=[row_spec, row_spec],
        out_shape=[
            jax.ShapeDtypeStruct((NPAD, C), _f32),
            jax.ShapeDtypeStruct((NPAD, C), _f32),
        ],
    )

  return t1, make_t23(False), make_t23(True)


@jax.jit
def kernel(points, features, neighbors, kernel_points, W_stem,
           W_a1, W_dw1, W_b1, W_c1, W_a2, W_dw2, W_b2, W_c2):
  gather2, gather1 = _make_sc_calls()
  t1, t2, t3 = _make_tc_calls()

  ftab = jnp.pad(features, ((0, NPAD - N), (0, 0)))
  ptab = jnp.pad(points, ((0, NPAD - N), (0, 125)))
  ctab = jnp.concatenate([ftab, ptab], axis=1)
  idx = jnp.pad(neighbors, ((0, NPAD - N), (0, 0))).reshape(MPAD)
  pts3 = jnp.pad(points, ((0, NPAD - N), (0, 0))).reshape(NPAD, 1, 3)
  kpt = jnp.pad(kernel_points.T, ((0, 5), (0, 1)))
  wdw1 = jnp.pad(W_dw1, ((0, 1), (0, 0)))
  wdw2 = jnp.pad(W_dw2, ((0, 1), (0, 0)))

  gc = gather2(ctab, idx).reshape(NPAD, K, 2 * C)

  x1, h1, infl = t1(gc, pts3, kpt, W_stem, W_a1)

  g1 = gather1(h1, idx).reshape(NPAD, K, C)
  x2, h2 = t2(g1, infl, x1, wdw1, W_b1, W_c1, W_a2)

  g2 = gather1(h2, idx).reshape(NPAD, K, C)
  x3, _ = t3(g2, infl, x2, wdw2, W_b2, W_c2, W_a2)

  return x3[:N]


# speedup vs baseline: 1.0352x; 1.0021x over previous
import functools

import jax
import jax.numpy as jnp
from jax import lax
from jax.experimental import pallas as pl
from jax.experimental.pallas import tpu as pltpu
from jax.experimental.pallas import tpu_sc as plsc

N = 10000
K = 32
KP = 15
C = 128
EXP = 4
SIGMA = 0.15

NPAD = 10240
B = 256
MPAD = NPAD * K
CH = 128
CH2 = 64

_f32 = jnp.float32


def _sc_gather1_body(tab, idx, out, idx_v, buf, gsems, *, nbuf, nc,
                     per_w, chunks, ch=CH):
  wid = lax.axis_index("s") * nc + lax.axis_index("c")
  base0 = wid * per_w
  pltpu.sync_copy(idx.at[pl.ds(base0, per_w)], idx_v)

  def issue(j, t):
    b = j * (nbuf * ch) + t * ch
    pltpu.async_copy(tab.at[idx_v.at[pl.ds(b, ch)]], buf.at[t], gsems[t])

  def wait_g(t):
    pltpu.make_async_copy(tab.at[idx_v.at[pl.ds(0, ch)]], buf.at[t],
                          gsems[t]).wait()

  def drain(j, t):
    b = j * (nbuf * ch) + t * ch
    pltpu.sync_copy(buf.at[t], out.at[pl.ds(base0 + b, ch)])

  def body(j, carry):
    for t in range(nbuf):
      wait_g(t)
      drain(j - 1, t)
      issue(j, t)
    return carry

  for t in range(nbuf):
    issue(0, t)
  lax.fori_loop(1, chunks // nbuf, body, 0)
  for t in range(nbuf):
    wait_g(t)
    drain(chunks // nbuf - 1, t)


def _make_sc_calls():
  info = plsc.get_sparse_core_info()
  nc, ns = info.num_cores, info.num_subcores
  per_w = MPAD // (nc * ns)
  chunks = per_w // CH
  mesh = plsc.VectorSubcoreMesh(core_axis_name="c", subcore_axis_name="s")

  def mk(body, width, nbuf, out_type, ch):
    n_chunks = per_w // ch
    assert n_chunks % nbuf == 0
    return pl.kernel(
        functools.partial(body, nbuf=nbuf, nc=nc, per_w=per_w,
                          chunks=n_chunks, ch=ch),
        mesh=mesh,
        out_type=out_type,
        scratch_types=[
            pltpu.VMEM((per_w,), jnp.int32),
            pltpu.VMEM((nbuf, ch, width), _f32),
            [pltpu.SemaphoreType.DMA] * nbuf,
        ],
    )

  gather2 = mk(_sc_gather1_body, 2 * C, 4,
               jax.ShapeDtypeStruct((MPAD, 2 * C), _f32), CH2)
  gather1 = mk(_sc_gather1_body, C, 5,
               jax.ShapeDtypeStruct((MPAD, C), _f32), CH)
  return gather2, gather1


def _leaky(x):
  return jnp.where(x >= 0, x, 0.1 * x)


def _infl_all(gp, ctr, kpt):
  dx = gp[:, :, 0:1] - ctr[:, :, 0:1]
  dy = gp[:, :, 1:2] - ctr[:, :, 1:2]
  dz = gp[:, :, 2:3] - ctr[:, :, 2:3]
  kx = kpt[0:1, :].reshape(1, 1, 16)
  ky = kpt[1:2, :].reshape(1, 1, 16)
  kz = kpt[2:3, :].reshape(1, 1, 16)
  ex = dx - kx
  ey = dy - ky
  ez = dz - kz
  d2 = ex * ex + ey * ey + ez * ez
  dist = jnp.sqrt(d2 + 1e-12)
  return jnp.maximum(1.0 - dist * (1.0 / SIGMA), 0.0)


def t1_body(gc_ref, pts_ref, kpt_ref, wst_ref, wa1_ref,
            x_ref, h1_ref, infl_ref):
  f = gc_ref[:, :, 0:C]
  gp = gc_ref[:, :, C:C + 16]
  infl = _infl_all(gp, pts_ref[...], kpt_ref[...])
  x = jnp.zeros((B, C), _f32)
  for p in range(KP):
    aggp = jnp.sum(f * infl[:, :, p:p + 1], axis=1)
    x = x + jnp.dot(aggp, wst_ref[p], preferred_element_type=_f32)
  x = _leaky(x)
  x_ref[...] = x
  h1_ref[...] = _leaky(jnp.dot(x, wa1_ref[...], preferred_element_type=_f32))
  infl_ref[...] = infl


def t23_body(gh_ref, infl_ref, x_ref, wdw_ref, wb_ref,
             wc_ref, wa_ref, x2_ref, h2_ref, *, last):
  infl = infl_ref[...]
  wedge = jnp.dot(infl.reshape(B * K, 16), wdw_ref[...],
                  preferred_element_type=_f32)
  g = gh_ref[...].reshape(B * K, C)
  h = jnp.sum((g * wedge).reshape(B, K, C), axis=1)
  h = _leaky(h)
  h = _leaky(jnp.dot(h, wb_ref[...], preferred_element_type=_f32))
  h = jnp.dot(h, wc_ref[...], preferred_element_type=_f32)
  x2 = x_ref[...] + h
  x2_ref[...] = x2
  if not last:
    h2_ref[...] = _leaky(jnp.dot(x2, wa_ref[...],
                                 preferred_element_type=_f32))


def _edge_spec():
  return pl.BlockSpec((B, K, C), lambda i: (i, 0, 0))


def _full(shape):
  return pl.BlockSpec(shape, lambda i: tuple(0 for _ in shape))


def _make_tc_calls():
  grid = (NPAD // B,)
  row_spec = pl.BlockSpec((B, C), lambda i: (i, 0))
  gp_spec = pl.BlockSpec((B, K, 16), lambda i: (i, 0, 0))
  pts_spec = pl.BlockSpec((B, 1, 3), lambda i: (i, 0, 0))

  t1 = pl.pallas_call(
      t1_body,
      grid=grid,
      in_specs=[
          pl.BlockSpec((B, K, 2 * C), lambda i: (i, 0, 0)), pts_spec,
          _full((8, 16)), _full((KP, C, C)), _full((C, C)),
      ],
      out_specs=[row_spec, row_spec, gp_spec],
      out_shape=[
          jax.ShapeDtypeStruct((NPAD, C), _f32),
          jax.ShapeDtypeStruct((NPAD, C), _f32),
          jax.ShapeDtypeStruct((NPAD, K, 16), _f32),
      ],
  )

  def make_t23(last):
    return pl.pallas_call(
        functools.partial(t23_body, last=last),
        grid=grid,
        in_specs=[
            _edge_spec(), gp_spec, row_spec,
            _full((16, C)), _full((C, EXP * C)), _full((EXP * C, C)),
            _full((C, C)),
        ],
        out_specs=[row_spec, row_spec],
        out_shape=[
            jax.ShapeDtypeStruct((NPAD, C), _f32),
            jax.ShapeDtypeStruct((NPAD, C), _f32),
        ],
    )

  return t1, make_t23(False), make_t23(True)


@jax.jit
def kernel(points, features, neighbors, kernel_points, W_stem,
           W_a1, W_dw1, W_b1, W_c1, W_a2, W_dw2, W_b2, W_c2):
  gather2, gather1 = _make_sc_calls()
  t1, t2, t3 = _make_tc_calls()

  ftab = jnp.pad(features, ((0, NPAD - N), (0, 0)))
  ptab = jnp.pad(points, ((0, NPAD - N), (0, 125)))
  ctab = jnp.concatenate([ftab, ptab], axis=1)
  idx = jnp.pad(neighbors, ((0, NPAD - N), (0, 0))).reshape(MPAD)
  pts3 = jnp.pad(points, ((0, NPAD - N), (0, 0))).reshape(NPAD, 1, 3)
  kpt = jnp.pad(kernel_points.T, ((0, 5), (0, 1)))
  wdw1 = jnp.pad(W_dw1, ((0, 1), (0, 0)))
  wdw2 = jnp.pad(W_dw2, ((0, 1), (0, 0)))

  gc = gather2(ctab, idx).reshape(NPAD, K, 2 * C)

  x1, h1, infl = t1(gc, pts3, kpt, W_stem, W_a1)

  g1 = gather1(h1, idx).reshape(NPAD, K, C)
  x2, h2 = t2(g1, infl, x1, wdw1, W_b1, W_c1, W_a2)

  g2 = gather1(h2, idx).reshape(NPAD, K, C)
  x3, _ = t3(g2, infl, x2, wdw2, W_b2, W_c2, W_a2)

  return x3[:N]
